# Initial kernel scaffold; baseline (speedup 1.0000x reference)
import jax
import jax.numpy as jnp
from jax.experimental import pallas as pl


def kernel(inputs, embeddings):
    raise NotImplementedError("write your pallas kernel here")


# speedup vs baseline: 7.7527x; 7.7527x over previous
import jax
import jax.numpy as jnp
from jax import lax
from jax.experimental import pallas as pl
from jax.experimental.pallas import tpu as pltpu
from jax.experimental.pallas import tpu_sc as plsc

B = 1024
L = 50
V = 21128
S = 32
D = S * S
NW = 32
B_PER_W = B // NW


def _glyph_body(idx_hbm, emb_hbm, out_hbm, idx_v, g_v, sem):
    wid = lax.axis_index("s") * 2 + lax.axis_index("c")
    base = wid * B_PER_W
    pltpu.sync_copy(idx_hbm.at[pl.ds(base, B_PER_W)], idx_v)

    def body(i, carry):
        b = base + i
        pltpu.async_copy(emb_hbm.at[idx_v.at[i]], g_v, sem).wait()
        for r in range(S):
            pltpu.sync_copy(g_v.at[:, pl.ds(r * S, S)], out_hbm.at[b, r])
        return carry

    lax.fori_loop(0, B_PER_W, body, 0)


def kernel(inputs, embeddings):
    emb2 = embeddings.reshape(V, D)
    mesh = plsc.VectorSubcoreMesh(core_axis_name="c", subcore_axis_name="s")
    out = pl.kernel(
        _glyph_body,
        out_type=jax.ShapeDtypeStruct((B, S, L, S), jnp.float32),
        mesh=mesh,
        scratch_types=[
            pltpu.VMEM((B_PER_W, L), jnp.int32),
            pltpu.VMEM((L, D), jnp.float32),
            pltpu.SemaphoreType.DMA,
        ],
        compiler_params=pltpu.CompilerParams(use_tc_tiling_on_sc=False),
    )(inputs, emb2)
    return out.reshape(B, S, L * S, 1)


# speedup vs baseline: 8.4534x; 1.0904x over previous
import jax
import jax.numpy as jnp
from jax import lax
from jax.experimental import pallas as pl
from jax.experimental.pallas import tpu as pltpu
from jax.experimental.pallas import tpu_sc as plsc

B = 1024
L = 50
V = 21128
S = 32
D = S * S
NW = 32
B_PER_W = B // NW


def _glyph_body(idx_hbm, emb_hbm, out_hbm, idx_v, g0, g1, gs0, gs1, ss):
    wid = lax.axis_index("s") * 2 + lax.axis_index("c")
    base = wid * B_PER_W
    pltpu.sync_copy(idx_hbm.at[pl.ds(base, B_PER_W)], idx_v)

    bufs = (g0, g1)
    gsems = (gs0, gs1)

    pltpu.async_copy(emb_hbm.at[idx_v.at[0]], g0, gs0)

    def outer(i, carry):
        for k in range(2):
            cur = 2 * i + k
            b = base + cur
            pltpu.make_async_copy(
                emb_hbm.at[idx_v.at[cur]], bufs[k], gsems[k]
            ).wait()
            nxt = cur + 1

            @pl.when(nxt < B_PER_W)
            def _():
                pltpu.async_copy(
                    emb_hbm.at[idx_v.at[nxt]], bufs[k ^ 1], gsems[k ^ 1]
                )

            cps = [
                pltpu.async_copy(
                    bufs[k].at[:, pl.ds(r * S, S)], out_hbm.at[b, r], ss
                )
                for r in range(S)
            ]
            for cp in cps:
                cp.wait()
        return carry

    lax.fori_loop(0, B_PER_W // 2, outer, 0)


def kernel(inputs, embeddings):
    emb2 = embeddings.reshape(V, D)
    mesh = plsc.VectorSubcoreMesh(core_axis_name="c", subcore_axis_name="s")
    out = pl.kernel(
        _glyph_body,
        out_type=jax.ShapeDtypeStruct((B, S, L, S), jnp.float32),
        mesh=mesh,
        scratch_types=[
            pltpu.VMEM((B_PER_W, L), jnp.int32),
            pltpu.VMEM((L, D), jnp.float32),
            pltpu.VMEM((L, D), jnp.float32),
            pltpu.SemaphoreType.DMA,
            pltpu.SemaphoreType.DMA,
            pltpu.SemaphoreType.DMA,
        ],
        compiler_params=pltpu.CompilerParams(use_tc_tiling_on_sc=False),
    )(inputs, emb2)
    return out.reshape(B, S, L * S, 1)
